# Initial kernel scaffold; baseline (speedup 1.0000x reference)
import jax
import jax.numpy as jnp
from jax.experimental import pallas as pl


def kernel(s0_diab_logits, s0_hr, s0_sysbp, s0_glucose, s0_percoxyg, policy_logits, u_diab, u_hr, u_sysbp, u_glucose, u_percoxyg, u_policy):
    raise NotImplementedError("write your pallas kernel here")


# speedup vs baseline: 3.2801x; 3.2801x over previous
import functools

import jax
import jax.numpy as jnp
from jax.experimental import pallas as pl
from jax.experimental.pallas import tpu as pltpu

_EPS = 1e-10


def _gmb(u):
    return -jnp.log(-jnp.log(u + _EPS) + _EPS)


def _body(dl_ref, pol_ref, t_hr, t_sbp, t_glu, t_po,
          ud_ref, u_hr, u_sbp, u_glu, u_po, up_ref,
          samples_ref, actions_ref, logp_ref,
          diab_s, *vs, V, C, N, B):
    groups = [tuple(vs[5 * k + j] for j in range(5)) for k in range(4)]
    i = pl.program_id(0)

    @pl.when(i == 0)
    def _init():
        dl = dl_ref[...]
        zd = dl + _gmb(ud_ref[...])
        s0 = (zd[:, 1:2] > zd[:, 0:1]).astype(jnp.int32)
        diab_s[...] = s0
        m2 = jnp.max(dl)
        lse2 = m2 + jnp.log(jnp.sum(jnp.exp(dl - m2)))
        samples_ref[:, 0:1] = s0
        logp_ref[:, 0:1] = jnp.where(s0 == 1, dl[0, 1], dl[0, 0]) - lse2

        pv = pol_ref[...]
        zp = pv + _gmb(up_ref[...])
        a = jnp.argmax(zp, axis=1).astype(jnp.int32)[:, None]
        actions_ref[...] = a
        mp = jnp.max(pv)
        lsep = mp + jnp.log(jnp.sum(jnp.exp(pv - mp)))
        ia8 = jax.lax.broadcasted_iota(jnp.int32, zp.shape, 1)
        tvp = jnp.sum(jnp.where(ia8 == a, pv, 0.0), axis=1, keepdims=True)
        logp_ref[:, 5:6] = tvp - lsep

        for (bz, bj, btv, m_s, s_s) in groups:
            bz[...] = jnp.full(bz.shape, -jnp.inf, jnp.float32)
            bj[...] = jnp.zeros(bj.shape, jnp.int32)
            btv[...] = jnp.zeros(btv.shape, jnp.float32)
            m_s[...] = jnp.full(m_s.shape, -jnp.inf, jnp.float32)
            s_s[...] = jnp.zeros(s_s.shape, jnp.float32)

    diab = diab_s[...]
    base = i * C
    col = jax.lax.broadcasted_iota(jnp.int32, (B, C), 1)
    valid = (base + col) < V
    col2 = jax.lax.broadcasted_iota(jnp.int32, (2, C), 1)
    valid2 = (base + col2) < V

    for (t_ref, u_ref), (bz, bj, btv, m_s, s_s) in zip(
            [(t_hr, u_hr), (t_sbp, u_sbp), (t_glu, u_glu), (t_po, u_po)],
            groups):
        g = _gmb(u_ref[...])
        t = t_ref[...]
        tsel = jnp.where(diab == 1, t[1:2, :], t[0:1, :])
        z = jnp.where(valid, tsel + g, -jnp.inf)
        cm = jnp.max(z, axis=1, keepdims=True)
        cj = jnp.argmax(z, axis=1).astype(jnp.int32)[:, None]
        tv = jnp.sum(jnp.where(col == cj, tsel, 0.0), axis=1, keepdims=True)
        upd = cm > bz[...]
        bj[...] = jnp.where(upd, cj + base, bj[...])
        btv[...] = jnp.where(upd, tv, btv[...])
        bz[...] = jnp.where(upd, cm, bz[...])
        tm = jnp.max(jnp.where(valid2, t, -jnp.inf), axis=1, keepdims=True)
        m_old = m_s[...]
        m_new = jnp.maximum(m_old, tm)
        se = jnp.sum(jnp.where(valid2, jnp.exp(t - m_new), 0.0),
                     axis=1, keepdims=True)
        s_s[...] = s_s[...] * jnp.exp(m_old - m_new) + se
        m_s[...] = m_new

    @pl.when(i == N - 1)
    def _fin():
        diab_f = diab_s[...]
        for v, (bz, bj, btv, m_s, s_s) in enumerate(groups):
            samples_ref[:, v + 1:v + 2] = bj[...]
            lse = m_s[...] + jnp.log(s_s[...])
            lse_sel = jnp.where(diab_f == 1, lse[1, 0], lse[0, 0])
            logp_ref[:, v + 1:v + 2] = btv[...] - lse_sel


def kernel(s0_diab_logits, s0_hr, s0_sysbp, s0_glucose, s0_percoxyg,
           policy_logits, u_diab, u_hr, u_sysbp, u_glucose, u_percoxyg,
           u_policy):
    B, V = u_hr.shape
    A = u_policy.shape[1]
    C = 4096
    N = pl.cdiv(V, C)
    dl = s0_diab_logits.reshape(1, 2)
    pol = policy_logits.reshape(1, A)

    const2 = lambda i: (0, 0)
    tspec = pl.BlockSpec((2, C), lambda i: (0, i))
    uspec = pl.BlockSpec((B, C), lambda i: (0, i))

    scratch = [pltpu.VMEM((B, 1), jnp.int32)]
    for _ in range(4):
        scratch += [pltpu.VMEM((B, 1), jnp.float32),
                    pltpu.VMEM((B, 1), jnp.int32),
                    pltpu.VMEM((B, 1), jnp.float32),
                    pltpu.VMEM((2, 1), jnp.float32),
                    pltpu.VMEM((2, 1), jnp.float32)]

    samples, actions2, logp = pl.pallas_call(
        functools.partial(_body, V=V, C=C, N=N, B=B),
        grid=(N,),
        in_specs=[
            pl.BlockSpec((1, 2), const2),
            pl.BlockSpec((1, A), const2),
            tspec, tspec, tspec, tspec,
            pl.BlockSpec((B, 2), const2),
            uspec, uspec, uspec, uspec,
            pl.BlockSpec((B, A), const2),
        ],
        out_specs=[
            pl.BlockSpec((B, 5), const2),
            pl.BlockSpec((B, 1), const2),
            pl.BlockSpec((B, 6), const2),
        ],
        out_shape=[
            jax.ShapeDtypeStruct((B, 5), jnp.int32),
            jax.ShapeDtypeStruct((B, 1), jnp.int32),
            jax.ShapeDtypeStruct((B, 6), jnp.float32),
        ],
        scratch_shapes=scratch,
        compiler_params=pltpu.CompilerParams(
            dimension_semantics=("arbitrary",)),
    )(dl, pol, s0_hr, s0_sysbp, s0_glucose, s0_percoxyg,
      u_diab, u_hr, u_sysbp, u_glucose, u_percoxyg, u_policy)

    return samples, actions2[:, 0], logp
